# baseline (speedup 1.0000x reference)
import functools

import jax
import jax.numpy as jnp
from jax import lax
from jax.experimental import pallas as pl
from jax.experimental.pallas import tpu as pltpu
from jax.experimental.pallas import tpu_sc as plsc

B = 16384
F = 26
K = 32
NV = 1000000
NC = 2
NS = 16
NW = NC * NS
BPW = B // NW
RPC = 4
IPC = RPC * F
NCHUNK = BPW // RPC
NBUF = 4
IPW = BPW * F
W1PAD = 112
VPB = 256
TRPB = VPB // 8
NSTEP = NV // VPB
NBLK = NV // 128
NR = 4


def _tr_body(vt_hbm, tail_hbm, trm_hbm, tbuf, obuf, tailbuf, *sems):
    sem_in = sems[:NR]
    sem_out = sems[NR:]
    wid = lax.axis_index("s") * NC + lax.axis_index("c")

    lane = lax.iota(jnp.int32, 16)
    ld4 = lane >> 2
    lm4 = lane & 3

    def in_copy(st, s):
        return pltpu.make_async_copy(
            vt_hbm.at[:, pl.ds(st * VPB, VPB)], tbuf.at[s], sem_in[s])

    def out_copy(st, s):
        return pltpu.make_async_copy(
            obuf.at[s], trm_hbm.at[pl.ds(st * TRPB, TRPB), :], sem_out[s])

    for s in range(NR):
        @pl.when(wid + s * NW < NSTEP)
        def _():
            in_copy(wid + s * NW, s).start()

    def step_group(i, carry):
        for s in range(NR):
            j = i * NR + s
            st = wid + j * NW

            @pl.when(st < NSTEP)
            def _():
                in_copy(st, s).wait()

                @pl.when(j >= NR)
                def _():
                    out_copy(st - NR * NW, s).wait()

                def vv_body(vv0, c):
                    v0 = vv0 * 4
                    dst_row = jnp.broadcast_to(
                        vv0 >> 1, (16,)).astype(jnp.int32)
                    col0 = ((vv0 & 1) << 6) + (lm4 << 4)
                    src_col = v0 + lm4
                    words = []
                    for j0 in range(0, K // 2, 4):
                        a = plsc.load_gather(tbuf.at[s], [j0 + ld4, src_col])
                        b = plsc.load_gather(
                            tbuf.at[s], [j0 + 16 + ld4, src_col])
                        p = plsc.pack(a, b, format=plsc.PackFormat.INTERLEAVED)
                        words.append(plsc.bitcast(p, jnp.int32))
                    for i, j0 in enumerate(range(0, K // 2, 4)):
                        plsc.store_scatter(
                            obuf.at[s], [dst_row, col0 + j0 + ld4], words[i])
                    return c

                lax.fori_loop(0, VPB // 4, vv_body, 0)
                out_copy(st, s).start()

                nxt = st + NR * NW

                @pl.when(nxt < NSTEP)
                def _():
                    in_copy(nxt, s).start()
        return carry

    lax.fori_loop(0, (NSTEP // NW + NR) // NR, step_group, 0)

    for s in range(NR):
        out_copy(0, s).wait()

    @pl.when(wid == 0)
    def _():
        pltpu.sync_copy(tail_hbm, tailbuf)
        pltpu.sync_copy(tailbuf, trm_hbm.at[pl.ds(NSTEP * TRPB, 8), :])


def _fm_body(idxt_hbm, w0_hbm, w1_hbm, trm_hbm, out_hbm,
             idxbt, idx_v, ridx_v, vrows, w1rows, outv, w0v, *sems):
    sem_v = sems[:NBUF]
    sem_w = sems[NBUF:]
    wid = lax.axis_index("s") * NC + lax.axis_index("c")

    pltpu.sync_copy(idxt_hbm.at[:, pl.ds(wid * BPW, BPW)], idxbt)
    pltpu.sync_copy(w0_hbm, w0v)

    lane = lax.iota(jnp.int32, 16)
    lane26 = lane * F

    def tr_idx(b0, carry):
        base = b0 * 16 * F
        for f in range(F):
            vec = idxbt[f, pl.ds(b0 * 16, 16)]
            dst = base + f + lane26
            plsc.store_scatter(idx_v, [dst], vec)
            plsc.store_scatter(ridx_v, [dst], vec >> 3)
        return carry

    lax.fori_loop(0, BPW // 16, tr_idx, 0)

    zero16 = jnp.zeros((16,), jnp.float32)
    for b in range(NBUF):
        w1rows[b, pl.ds(96, 16)] = zero16

    m10 = jnp.where(lane < 10, 1.0, 0.0).astype(jnp.float32)
    w0s = w0v[pl.ds(0, 16)][0]
    out_mask = lane < RPC
    lane_mod = lane & (RPC - 1)

    def v_copy(g, b):
        return pltpu.make_async_copy(
            trm_hbm.at[ridx_v.at[pl.ds(g * IPC, IPC)]], vrows.at[b], sem_v[b])

    def w_copy(g, b):
        return pltpu.make_async_copy(
            w1_hbm.at[idx_v.at[pl.ds(g * IPC, IPC)]],
            w1rows.at[b, pl.ds(0, IPC)], sem_w[b])

    for b in range(NBUF):
        v_copy(b, b).start()
        w_copy(b, b).start()

    def chunk_body(i, carry):
        g0 = i * NBUF
        for b in range(NBUF):
            g = g0 + b
            v_copy(g, b).wait()
            w_copy(g, b).wait()
            offs = []
            for k in range(7):
                ivec = idx_v[pl.ds(g * IPC + k * 16, 16)]
                offs.append((ivec & 7) << 4)
            vals = zero16
            for r in range(RPC):
                o = r * F
                acc = None
                for f in range(F):
                    j = o + f
                    oj = offs[j // 16][j % 16]
                    xw = vrows[b, j, pl.ds(oj, 16)]
                    x0, x1 = plsc.unpack(
                        plsc.bitcast(xw, jnp.bfloat16),
                        format=plsc.PackFormat.INTERLEAVED)
                    if acc is None:
                        s0, s1 = x0, x1
                        q0, q1 = x0 * x0, x1 * x1
                        acc = True
                    else:
                        s0 += x0
                        s1 += x1
                        q0 += x0 * x0
                        q1 += x1 * x1
                t = s0 * s0 + s1 * s1 - q0 - q1
                la = w1rows[b, pl.ds(F * r, 16)]
                lb = w1rows[b, pl.ds(F * r + 16, 16)] * m10
                val = jnp.sum(0.5 * t + la + lb) + w0s
                vals = jnp.where(lane == r, val, vals)
            plsc.store_scatter(outv, [g * RPC + lane_mod], vals, mask=out_mask)
            nxt = g + NBUF

            @pl.when(nxt < NCHUNK)
            def _():
                v_copy(nxt, b).start()
                w_copy(nxt, b).start()
        return carry

    lax.fori_loop(0, NCHUNK // NBUF, chunk_body, 0)
    pltpu.sync_copy(outv, out_hbm.at[pl.ds(wid * BPW, BPW)])


@jax.jit
def _fm(idxt, w0b, w1f, vt, tail):
    mesh = plsc.VectorSubcoreMesh(core_axis_name="c", subcore_axis_name="s")
    params = pltpu.CompilerParams(
        needs_layout_passes=False, use_tc_tiling_on_sc=True)

    transpose = functools.partial(
        pl.kernel,
        out_type=jax.ShapeDtypeStruct((NV // 8, 128), jnp.int32),
        mesh=mesh,
        scratch_types=[
            pltpu.VMEM((NR, K, VPB), jnp.float32),
            pltpu.VMEM((NR, TRPB, 128), jnp.int32),
            pltpu.VMEM((8, 128), jnp.int32),
        ] + [pltpu.SemaphoreType.DMA] * (2 * NR),
        compiler_params=params,
    )(_tr_body)
    trm = transpose(vt, tail)

    fm = functools.partial(
        pl.kernel,
        out_type=jax.ShapeDtypeStruct((B,), jnp.float32),
        mesh=mesh,
        scratch_types=[
            pltpu.VMEM((F, BPW), jnp.int32),
            pltpu.VMEM((IPW + 16,), jnp.int32),
            pltpu.VMEM((IPW,), jnp.int32),
            pltpu.VMEM((NBUF, IPC, 128), jnp.int32),
            pltpu.VMEM((NBUF, W1PAD), jnp.float32),
            pltpu.VMEM((BPW,), jnp.float32),
            pltpu.VMEM((16,), jnp.float32),
        ] + [pltpu.SemaphoreType.DMA] * (2 * NBUF),
        compiler_params=params,
    )(_fm_body)
    return fm(idxt, w0b, w1f, trm)


def kernel(inputs, W0, W1, V):
    idxt = inputs.astype(jnp.int32).T
    w0b = jnp.broadcast_to(W0.astype(jnp.float32), (16,))
    w1f = W1.reshape(-1)
    vt = V.T
    tv = V[NBLK * 128:]
    tail = jax.lax.bitcast_convert_type(
        jnp.stack([tv[:, :16], tv[:, 16:]], axis=-1).astype(jnp.bfloat16),
        jnp.int32).reshape(8, 128)
    out = _fm(idxt, w0b, w1f, vt, tail)
    return out.reshape(B, 1)


# speedup vs baseline: 1.2074x; 1.2074x over previous
import functools

import jax
import jax.numpy as jnp
from jax import lax
from jax.experimental import pallas as pl
from jax.experimental.pallas import tpu as pltpu
from jax.experimental.pallas import tpu_sc as plsc

B = 16384
F = 26
K = 32
NV = 1000000
NC = 2
NS = 16
NW = NC * NS
BPW = B // NW
RPC = 4
IPC = RPC * F
NCHUNK = BPW // RPC
NBUF = 4
IPW = BPW * F
W1PAD = 112
VPB = 256
TRPB = VPB // 4
NSTEP = NV // VPB
NBLK = NV // 128
NR = 6


def _tr_body(vt_hbm, tail_hbm, trm_hbm, tbuf, obuf, tailbuf, *sems):
    sem_in = sems[:NR]
    sem_out = sems[NR:]
    wid = lax.axis_index("s") * NC + lax.axis_index("c")

    lane = lax.iota(jnp.int32, 16)
    ld4 = lane >> 2
    lm4 = lane & 3
    dst_col = (lm4 << 5) + ld4

    def in_copy(st, s):
        return pltpu.make_async_copy(
            vt_hbm.at[:, pl.ds(st * VPB, VPB)], tbuf.at[s], sem_in[s])

    def out_copy(st, s):
        return pltpu.make_async_copy(
            obuf.at[s], trm_hbm.at[pl.ds(st * TRPB, TRPB), :], sem_out[s])

    for s in range(NR):
        @pl.when(wid + s * NW < NSTEP)
        def _():
            in_copy(wid + s * NW, s).start()

    def step_group(i, carry):
        for s in range(NR):
            j = i * NR + s
            st = wid + j * NW

            @pl.when(st < NSTEP)
            def _():
                in_copy(st, s).wait()

                @pl.when(j >= NR)
                def _():
                    out_copy(st - NR * NW, s).wait()

                def vv_body(vv0, c):
                    v0 = vv0 * 4
                    dst_row = jnp.broadcast_to(vv0, (16,)).astype(jnp.int32)
                    src_col = v0 + lm4
                    vecs = []
                    for k0 in range(0, K, 4):
                        vecs.append(plsc.load_gather(
                            tbuf.at[s], [k0 + ld4, src_col]))
                    for i, k0 in enumerate(range(0, K, 4)):
                        plsc.store_scatter(
                            obuf.at[s], [dst_row, k0 + dst_col], vecs[i])
                    return c

                lax.fori_loop(0, VPB // 4, vv_body, 0)
                out_copy(st, s).start()

                nxt = st + NR * NW

                @pl.when(nxt < NSTEP)
                def _():
                    in_copy(nxt, s).start()
        return carry

    lax.fori_loop(0, (NSTEP // NW + NR) // NR, step_group, 0)

    for s in range(NR):
        out_copy(0, s).wait()

    @pl.when(wid == 0)
    def _():
        pltpu.sync_copy(tail_hbm, tailbuf)
        pltpu.sync_copy(tailbuf, trm_hbm.at[pl.ds(NBLK * 32, 16), :])


def _fm_body(idxt_hbm, w0_hbm, w1_hbm, trm_hbm, out_hbm,
             idxbt, idx_v, ridx_v, vrows, w1rows, outv, w0v, *sems):
    sem_v = sems[:NBUF]
    sem_w = sems[NBUF:]
    wid = lax.axis_index("s") * NC + lax.axis_index("c")

    pltpu.sync_copy(idxt_hbm.at[:, pl.ds(wid * BPW, BPW)], idxbt)
    pltpu.sync_copy(w0_hbm, w0v)

    lane = lax.iota(jnp.int32, 16)
    lane26 = lane * F

    def tr_idx(b0, carry):
        base = b0 * 16 * F
        for f in range(F):
            vec = idxbt[f, pl.ds(b0 * 16, 16)]
            dst = base + f + lane26
            plsc.store_scatter(idx_v, [dst], vec)
            plsc.store_scatter(ridx_v, [dst], vec >> 2)
        return carry

    lax.fori_loop(0, BPW // 16, tr_idx, 0)

    zero16 = jnp.zeros((16,), jnp.float32)
    for b in range(NBUF):
        w1rows[b, pl.ds(96, 16)] = zero16

    m10 = jnp.where(lane < 10, 1.0, 0.0).astype(jnp.float32)
    w0s = w0v[pl.ds(0, 16)][0]
    out_mask = lane < RPC
    lane_mod = lane & (RPC - 1)

    def v_copy(g, b):
        return pltpu.make_async_copy(
            trm_hbm.at[ridx_v.at[pl.ds(g * IPC, IPC)]], vrows.at[b], sem_v[b])

    def w_copy(g, b):
        return pltpu.make_async_copy(
            w1_hbm.at[idx_v.at[pl.ds(g * IPC, IPC)]],
            w1rows.at[b, pl.ds(0, IPC)], sem_w[b])

    for b in range(NBUF):
        v_copy(b, b).start()
        w_copy(b, b).start()

    def chunk_body(i, carry):
        g0 = i * NBUF
        for b in range(NBUF):
            g = g0 + b
            v_copy(g, b).wait()
            w_copy(g, b).wait()
            offs = []
            for k in range(7):
                ivec = idx_v[pl.ds(g * IPC + k * 16, 16)]
                offs.append((ivec & 3) << 5)
            vals = zero16
            for r in range(RPC):
                o = r * F
                acc = None
                for f in range(F):
                    j = o + f
                    oj = offs[j // 16][j % 16]
                    x0 = vrows[b, j, pl.ds(oj, 16)]
                    x1 = vrows[b, j, pl.ds(oj + 16, 16)]
                    if acc is None:
                        s0, s1 = x0, x1
                        q0, q1 = x0 * x0, x1 * x1
                        acc = True
                    else:
                        s0 += x0
                        s1 += x1
                        q0 += x0 * x0
                        q1 += x1 * x1
                t = s0 * s0 + s1 * s1 - q0 - q1
                la = w1rows[b, pl.ds(F * r, 16)]
                lb = w1rows[b, pl.ds(F * r + 16, 16)] * m10
                val = jnp.sum(0.5 * t + la + lb) + w0s
                vals = jnp.where(lane == r, val, vals)
            plsc.store_scatter(outv, [g * RPC + lane_mod], vals, mask=out_mask)
            nxt = g + NBUF

            @pl.when(nxt < NCHUNK)
            def _():
                v_copy(nxt, b).start()
                w_copy(nxt, b).start()
        return carry

    lax.fori_loop(0, NCHUNK // NBUF, chunk_body, 0)
    pltpu.sync_copy(outv, out_hbm.at[pl.ds(wid * BPW, BPW)])


@jax.jit
def _fm(idxt, w0b, w1f, vt, tail):
    mesh = plsc.VectorSubcoreMesh(core_axis_name="c", subcore_axis_name="s")
    params = pltpu.CompilerParams(
        needs_layout_passes=False, use_tc_tiling_on_sc=True)

    transpose = functools.partial(
        pl.kernel,
        out_type=jax.ShapeDtypeStruct((NV // 4, 128), jnp.float32),
        mesh=mesh,
        scratch_types=[
            pltpu.VMEM((NR, K, VPB), jnp.float32),
            pltpu.VMEM((NR, TRPB, 128), jnp.float32),
            pltpu.VMEM((16, 128), jnp.float32),
        ] + [pltpu.SemaphoreType.DMA] * (2 * NR),
        compiler_params=params,
    )(_tr_body)
    trm = transpose(vt, tail)

    fm = functools.partial(
        pl.kernel,
        out_type=jax.ShapeDtypeStruct((B,), jnp.float32),
        mesh=mesh,
        scratch_types=[
            pltpu.VMEM((F, BPW), jnp.int32),
            pltpu.VMEM((IPW + 16,), jnp.int32),
            pltpu.VMEM((IPW,), jnp.int32),
            pltpu.VMEM((NBUF, IPC, 128), jnp.float32),
            pltpu.VMEM((NBUF, W1PAD), jnp.float32),
            pltpu.VMEM((BPW,), jnp.float32),
            pltpu.VMEM((16,), jnp.float32),
        ] + [pltpu.SemaphoreType.DMA] * (2 * NBUF),
        compiler_params=params,
    )(_fm_body)
    return fm(idxt, w0b, w1f, trm)


def kernel(inputs, W0, W1, V):
    idxt = inputs.astype(jnp.int32).T
    w0b = jnp.broadcast_to(W0.astype(jnp.float32), (16,))
    w1f = W1.reshape(-1)
    vt = V.T
    tail = V[NBLK * 128:].reshape(16, 128)
    out = _fm(idxt, w0b, w1f, vt, tail)
    return out.reshape(B, 1)


# speedup vs baseline: 1.2139x; 1.0054x over previous
import functools

import jax
import jax.numpy as jnp
from jax import lax
from jax.experimental import pallas as pl
from jax.experimental.pallas import tpu as pltpu
from jax.experimental.pallas import tpu_sc as plsc

B = 16384
F = 26
K = 32
NV = 1000000
NC = 2
NS = 16
NW = NC * NS
BPW = B // NW
RPC = 4
IPC = RPC * F
NCHUNK = BPW // RPC
NBUF = 4
IPW = BPW * F
W1PAD = 112
VPB = 256
TRPB = VPB // 4
NSTEP = NV // VPB
NBLK = NV // 128
NR = 4


def _tr_body(vt_hbm, tail_hbm, trm_hbm, tbuf, obuf, tailbuf, *sems):
    sem_in = sems[:NR]
    sem_out = sems[NR:]
    wid = lax.axis_index("s") * NC + lax.axis_index("c")

    lane = lax.iota(jnp.int32, 16)
    ld4 = lane >> 2
    lm4 = lane & 3
    dst_col = (lm4 << 5) + ld4

    def in_copy(st, s):
        return pltpu.make_async_copy(
            vt_hbm.at[:, pl.ds(st * VPB, VPB)], tbuf.at[s], sem_in[s])

    def out_copy(st, s):
        return pltpu.make_async_copy(
            obuf.at[s], trm_hbm.at[pl.ds(st * TRPB, TRPB), :], sem_out[s])

    for s in range(NR):
        @pl.when(wid + s * NW < NSTEP)
        def _():
            in_copy(wid + s * NW, s).start()

    def step_group(i, carry):
        for s in range(NR):
            j = i * NR + s
            st = wid + j * NW

            @pl.when(st < NSTEP)
            def _():
                in_copy(st, s).wait()

                @pl.when(j >= NR)
                def _():
                    out_copy(st - NR * NW, s).wait()

                def vv_body(vv0, c):
                    v0 = vv0 * 4
                    dst_row = jnp.broadcast_to(vv0, (16,)).astype(jnp.int32)
                    src_col = v0 + lm4
                    vecs = []
                    for k0 in range(0, K, 4):
                        vecs.append(plsc.load_gather(
                            tbuf.at[s], [k0 + ld4, src_col]))
                    for i, k0 in enumerate(range(0, K, 4)):
                        plsc.store_scatter(
                            obuf.at[s], [dst_row, k0 + dst_col], vecs[i])
                    return c

                lax.fori_loop(0, VPB // 4, vv_body, 0)
                out_copy(st, s).start()

                nxt = st + NR * NW

                @pl.when(nxt < NSTEP)
                def _():
                    in_copy(nxt, s).start()
        return carry

    lax.fori_loop(0, (NSTEP // NW + NR) // NR, step_group, 0)

    for s in range(NR):
        out_copy(0, s).wait()

    @pl.when(wid == 0)
    def _():
        pltpu.sync_copy(tail_hbm, tailbuf)
        pltpu.sync_copy(tailbuf, trm_hbm.at[pl.ds(NBLK * 32, 16), :])


def _fm_body(idxt_hbm, w0_hbm, w1_hbm, trm_hbm, out_hbm,
             idxbt, idx_v, ridx_v, vrows, w1rows, outv, w0v, *sems):
    sem_v = sems[:NBUF]
    sem_w = sems[NBUF:]
    wid = lax.axis_index("s") * NC + lax.axis_index("c")

    pltpu.sync_copy(idxt_hbm.at[:, pl.ds(wid * BPW, BPW)], idxbt)
    pltpu.sync_copy(w0_hbm, w0v)

    lane = lax.iota(jnp.int32, 16)
    lane26 = lane * F

    def tr_idx(b0, carry):
        base = b0 * 16 * F
        for f in range(F):
            vec = idxbt[f, pl.ds(b0 * 16, 16)]
            dst = base + f + lane26
            plsc.store_scatter(idx_v, [dst], vec)
            plsc.store_scatter(ridx_v, [dst], vec >> 2)
        return carry

    lax.fori_loop(0, BPW // 16, tr_idx, 0)

    zero16 = jnp.zeros((16,), jnp.float32)
    for b in range(NBUF):
        w1rows[b, pl.ds(96, 16)] = zero16

    m10 = jnp.where(lane < 10, 1.0, 0.0).astype(jnp.float32)
    w0s = w0v[pl.ds(0, 16)][0]
    out_mask = lane < RPC
    lane_mod = lane & (RPC - 1)

    def v_copy(g, b):
        return pltpu.make_async_copy(
            trm_hbm.at[ridx_v.at[pl.ds(g * IPC, IPC)]], vrows.at[b], sem_v[b])

    def w_copy(g, b):
        return pltpu.make_async_copy(
            w1_hbm.at[idx_v.at[pl.ds(g * IPC, IPC)]],
            w1rows.at[b, pl.ds(0, IPC)], sem_w[b])

    for b in range(NBUF):
        v_copy(b, b).start()
        w_copy(b, b).start()

    def chunk_body(i, carry):
        g0 = i * NBUF
        for b in range(NBUF):
            g = g0 + b
            v_copy(g, b).wait()
            w_copy(g, b).wait()
            offs = []
            for k in range(7):
                ivec = idx_v[pl.ds(g * IPC + k * 16, 16)]
                offs.append((ivec & 3) << 5)
            vals = zero16
            for r in range(RPC):
                o = r * F
                acc = None
                for f in range(F):
                    j = o + f
                    oj = offs[j // 16][j % 16]
                    x0 = vrows[b, j, pl.ds(oj, 16)]
                    x1 = vrows[b, j, pl.ds(oj + 16, 16)]
                    if acc is None:
                        s0, s1 = x0, x1
                        q0, q1 = x0 * x0, x1 * x1
                        acc = True
                    else:
                        s0 += x0
                        s1 += x1
                        q0 += x0 * x0
                        q1 += x1 * x1
                t = s0 * s0 + s1 * s1 - q0 - q1
                la = w1rows[b, pl.ds(F * r, 16)]
                lb = w1rows[b, pl.ds(F * r + 16, 16)] * m10
                val = jnp.sum(0.5 * t + la + lb) + w0s
                vals = jnp.where(lane == r, val, vals)
            plsc.store_scatter(outv, [g * RPC + lane_mod], vals, mask=out_mask)
            nxt = g + NBUF

            @pl.when(nxt < NCHUNK)
            def _():
                v_copy(nxt, b).start()
                w_copy(nxt, b).start()
        return carry

    lax.fori_loop(0, NCHUNK // NBUF, chunk_body, 0)
    pltpu.sync_copy(outv, out_hbm.at[pl.ds(wid * BPW, BPW)])


@jax.jit
def _fm(idxt, w0b, w1f, vt, tail):
    mesh = plsc.VectorSubcoreMesh(core_axis_name="c", subcore_axis_name="s")
    params = pltpu.CompilerParams(
        needs_layout_passes=False, use_tc_tiling_on_sc=True)

    transpose = functools.partial(
        pl.kernel,
        out_type=jax.ShapeDtypeStruct((NV // 4, 128), jnp.float32),
        mesh=mesh,
        scratch_types=[
            pltpu.VMEM((NR, K, VPB), jnp.float32),
            pltpu.VMEM((NR, TRPB, 128), jnp.float32),
            pltpu.VMEM((16, 128), jnp.float32),
        ] + [pltpu.SemaphoreType.DMA] * (2 * NR),
        compiler_params=params,
    )(_tr_body)
    trm = transpose(vt, tail)

    fm = functools.partial(
        pl.kernel,
        out_type=jax.ShapeDtypeStruct((B,), jnp.float32),
        mesh=mesh,
        scratch_types=[
            pltpu.VMEM((F, BPW), jnp.int32),
            pltpu.VMEM((IPW + 16,), jnp.int32),
            pltpu.VMEM((IPW,), jnp.int32),
            pltpu.VMEM((NBUF, IPC, 128), jnp.float32),
            pltpu.VMEM((NBUF, W1PAD), jnp.float32),
            pltpu.VMEM((BPW,), jnp.float32),
            pltpu.VMEM((16,), jnp.float32),
        ] + [pltpu.SemaphoreType.DMA] * (2 * NBUF),
        compiler_params=params,
    )(_fm_body)
    return fm(idxt, w0b, w1f, trm)


def kernel(inputs, W0, W1, V):
    idxt = inputs.astype(jnp.int32).T
    w0b = jnp.broadcast_to(W0.astype(jnp.float32), (16,))
    w1f = W1.reshape(-1)
    vt = V.T
    tail = V[NBLK * 128:].reshape(16, 128)
    out = _fm(idxt, w0b, w1f, vt, tail)
    return out.reshape(B, 1)
